# baseline (speedup 1.0000x reference)
import functools

import jax
import jax.numpy as jnp
from jax import lax
from jax.experimental import pallas as pl
from jax.experimental.pallas import tpu as pltpu
from jax.experimental.pallas import tpu_sc as plsc

_SIZE = 256
_NFACES = 256
_TEXW = 512
_P = _SIZE * _SIZE

_ROWS_PER_STEP = 32
_GRID = _SIZE // _ROWS_PER_STEP

_NW = 32
_PPW = _P // _NW
_CHUNK = 128
_NCHUNK = _PPW // _CHUNK


def _lane_shuffle(s, idx):
    return lax.gather(
        s, idx[:, None],
        dimension_numbers=lax.GatherDimensionNumbers(
            offset_dims=(), collapsed_slice_dims=(0,), start_index_map=(0,)),
        slice_sizes=(1,),
        mode=lax.GatherScatterMode.PROMISE_IN_BOUNDS)


def _area2d(ax, ay, bx, by, cx, cy):
    return (bx - ax) * (cy - ay) - (by - ay) * (cx - ax)


def _raster_body(td, nv, zinit, px_ref, py_ref,
                 idx0, w0, w1, w2, w3, alpha):
    px = px_ref[...]
    py = py_ref[...]
    shp = px.shape
    pinf = jnp.full(shp, jnp.inf, jnp.float32)
    zero = jnp.zeros(shp, jnp.float32)

    def body(f, carry):
        best_k, best_u, best_v = carry
        x0 = td[f, 0]
        y0 = td[f, 1]
        x1 = td[f, 2]
        y1 = td[f, 3]
        x2 = td[f, 4]
        y2 = td[f, 5]
        u0 = td[f, 6]
        u1 = td[f, 7]
        u2 = td[f, 8]
        v0 = td[f, 9]
        v1 = td[f, 10]
        v2 = td[f, 11]
        zi0 = td[f, 12]
        zi1 = td[f, 13]
        zi2 = td[f, 14]
        inv_a = td[f, 15]
        pab = (px - x1) * (y0 - y1) - (py - y1) * (x0 - x1)
        pcb = (px - x2) * (y1 - y2) - (py - y2) * (x1 - x2)
        pca = (px - x0) * (y2 - y0) - (py - y0) * (x2 - x0)
        cover = (jnp.maximum(pab, 0.0) * jnp.maximum(pcb, 0.0)
                 * jnp.maximum(pca, 0.0)) > 0
        bw1 = pcb * inv_a
        bw2 = pca * inv_a
        bw3 = 1.0 - bw1 - bw2
        zinv = bw1 * zi0 + bw2 * zi1 + bw3 * zi2
        key = jnp.where(cover, zinv, jnp.inf)
        ui = bw1 * u0 + bw2 * u1 + bw3 * u2
        vi = bw1 * v0 + bw2 * v1 + bw3 * v2
        upd = key <= best_k
        return (jnp.where(upd, key, best_k),
                jnp.where(upd, ui, best_u),
                jnp.where(upd, vi, best_v))

    best_k, best_u, best_v = lax.fori_loop(
        0, nv[0, 0], body, (pinf, zero, zero))

    hit = best_k < jnp.inf
    best_z = jnp.where(hit, 1.0 / jnp.where(hit, best_k, 1.0), -jnp.inf)
    write = best_z >= zinit[0, 0]
    denom = jnp.where(write, best_k, 1.0)
    ptsz = 1.0 / denom
    uf = best_u * ptsz
    vf = best_v * ptsz

    x = (uf + 1.0) * 512.0 / 2.0 - 0.5
    y = (vf + 1.0) * 512.0 / 2.0 - 0.5
    x0f = jnp.floor(x)
    y0f = jnp.floor(y)
    wx1 = x - x0f
    wx0 = 1.0 - wx1
    wy1 = y - y0f
    wy0 = 1.0 - wy1

    def tap(xi, yi, wt, w_ref):
        validt = ((xi >= 0) & (xi <= _TEXW - 1)
                  & (yi >= 0) & (yi <= _TEXW - 1))
        w_ref[...] = jnp.where(write, wt * validt.astype(jnp.float32), 0.0)

    tap(x0f, y0f, wx0 * wy0, w0)
    tap(x0f + 1.0, y0f, wx1 * wy0, w1)
    tap(x0f, y0f + 1.0, wx0 * wy1, w2)
    tap(x0f + 1.0, y0f + 1.0, wx1 * wy1, w3)

    bx = jnp.clip(x0f, -1.0, 511.0)
    by = jnp.clip(y0f, -1.0, 511.0)
    bxi = jnp.clip((bx + 1.0).astype(jnp.int32), 0, _TEXW)
    byi = jnp.clip((by + 1.0).astype(jnp.int32), 0, _TEXW)
    idx0[...] = byi * (_TEXW + 1) + bxi
    alpha[...] = write.astype(jnp.float32)


def _rasterize(td, nv, zinit, px, py):
    bspec = pl.BlockSpec((_ROWS_PER_STEP, _SIZE), lambda i: (i, 0))
    sspec = pl.BlockSpec(memory_space=pltpu.SMEM)
    i32 = jax.ShapeDtypeStruct((_SIZE, _SIZE), jnp.int32)
    f32 = jax.ShapeDtypeStruct((_SIZE, _SIZE), jnp.float32)
    return pl.pallas_call(
        _raster_body,
        grid=(_GRID,),
        in_specs=[sspec, sspec, sspec, bspec, bspec],
        out_specs=[bspec] * 6,
        out_shape=[i32, f32, f32, f32, f32, f32],
    )(td, nv, zinit, px, py)


@functools.lru_cache(maxsize=1)
def _make_sc_sampler():
    mesh = plsc.VectorSubcoreMesh(core_axis_name="c", subcore_axis_name="s",
                                  num_cores=2, num_subcores=16)

    @functools.partial(
        pl.kernel,
        out_type=jax.ShapeDtypeStruct((3, _NW, _PPW), jnp.float32),
        mesh=mesh,
        scratch_types=[
            pltpu.VMEM((_NCHUNK, _CHUNK), jnp.int32),
            pltpu.VMEM((4 * _PPW,), jnp.float32),
            pltpu.VMEM((_PPW, 16), jnp.float32),
            pltpu.VMEM((3 * _PPW,), jnp.float32),
            pltpu.SemaphoreType.DMA,
        ],
        compiler_params=pltpu.CompilerParams(use_tc_tiling_on_sc=False,
                                             needs_layout_passes=False),
    )
    def sampler(tex_hbm, idx_hbm, w_hbm, out_hbm, idxs, ws, rows, outb, sem):
        wid = lax.axis_index("s") * 2 + lax.axis_index("c")
        pltpu.sync_copy(idx_hbm.at[wid], idxs)
        copies = []
        for c in range(_NCHUNK):
            copies.append(pltpu.async_copy(
                tex_hbm.at[idxs.at[c]],
                rows.at[pl.ds(c * _CHUNK, _CHUNK)], sem))
        pltpu.sync_copy(w_hbm.at[wid], ws)
        for cp in copies:
            cp.wait()

        lane = lax.iota(jnp.int32, 16)
        rot4 = lax.rem(lane + 4, 16)
        rot8 = lax.rem(lane + 8, 16)
        rot12 = lax.rem(lane + 12, 16)
        out_off = lane * _PPW
        out_mask = lane < 3
        woff = jax.lax.shift_right_logical(lane, 2) * _PPW

        def accum(i, _):
            for j in range(4):
                p = i * 4 + j
                s = rows[p, :] * plsc.load_gather(ws, [woff + p])
                r = s + _lane_shuffle(s, rot4)
                r = r + _lane_shuffle(s, rot8)
                r = r + _lane_shuffle(s, rot12)
                plsc.store_scatter(outb, [out_off + p], r, mask=out_mask)
            return 0

        lax.fori_loop(0, _PPW // 4, accum, 0)
        for ch in range(3):
            pltpu.sync_copy(outb.at[pl.ds(ch * _PPW, _PPW)],
                            out_hbm.at[ch, wid])

    return sampler


def kernel(vertices, faces, uv, uvfaces, uvmap):
    tris = vertices[faces]
    z_inv = 1.0 / tris[:, :, 2]
    uv2 = uv * 2.0 - 1.0
    uvs = uv2[uvfaces] * z_inv[..., None]
    t2 = tris[:, :, :2]
    area = _area2d(t2[:, 0, 0], t2[:, 0, 1], t2[:, 1, 0], t2[:, 1, 1],
                   t2[:, 2, 0], t2[:, 2, 1])
    normals = jnp.cross(tris[:, 1] - tris[:, 0], tris[:, 2] - tris[:, 0])
    normals = normals / jnp.linalg.norm(normals)
    bf = (normals @ jnp.array([0.0, 0.0, 1.0])) > 0
    valid = bf & (area >= 1e-9)
    a_s = jnp.where(jnp.abs(area) > 1e-12, area, 1.0)
    z_init = jnp.min(vertices[:, 2])

    td = jnp.stack([
        t2[:, 0, 0], t2[:, 0, 1], t2[:, 1, 0], t2[:, 1, 1],
        t2[:, 2, 0], t2[:, 2, 1],
        uvs[:, 0, 0], uvs[:, 1, 0], uvs[:, 2, 0],
        uvs[:, 0, 1], uvs[:, 1, 1], uvs[:, 2, 1],
        z_inv[:, 0], z_inv[:, 1], z_inv[:, 2],
        1.0 / a_s,
    ], axis=1)
    order = jnp.argsort(jnp.logical_not(valid), stable=True)
    td = td[order]
    nv = jnp.sum(valid.astype(jnp.int32)).reshape(1, 1)
    zinit = z_init.reshape(1, 1)

    lin = jnp.linspace(-1.0, 1.0, _SIZE)
    xx, yy = jnp.meshgrid(lin, lin, indexing='ij')
    pts = jnp.rot90(jnp.stack([xx, yy], axis=-1), 1)
    px = pts[..., 0]
    py = pts[..., 1]

    ib, w0, w1, w2, w3, alpha = _rasterize(td, nv, zinit, px, py)

    ext = jnp.pad(uvmap, ((0, 0), (1, 1), (1, 1)), mode='edge')
    n1 = _TEXW + 1
    zz = jnp.zeros((n1 * n1,), jnp.float32)
    slabs = []
    for dy, dx in ((0, 0), (0, 1), (1, 0), (1, 1)):
        sub = ext[:, dy:dy + n1, dx:dx + n1]
        slabs += [sub[0].reshape(-1), sub[1].reshape(-1),
                  sub[2].reshape(-1), zz]
    tex = jnp.stack(slabs, -1)

    idx_sc = ib.reshape(_NW, _NCHUNK, _CHUNK)
    w_sc = jnp.stack([w0.reshape(_NW, _PPW), w1.reshape(_NW, _PPW),
                      w2.reshape(_NW, _PPW), w3.reshape(_NW, _PPW)],
                     axis=1).reshape(_NW, 4 * _PPW)

    rgb = _make_sc_sampler()(tex, idx_sc, w_sc).reshape(3, _SIZE, _SIZE)
    return jnp.concatenate([rgb, alpha.reshape(1, _SIZE, _SIZE)], axis=0)


# speedup vs baseline: 1.4418x; 1.4418x over previous
import functools

import jax
import jax.numpy as jnp
from jax import lax
from jax.experimental import pallas as pl
from jax.experimental.pallas import tpu as pltpu
from jax.experimental.pallas import tpu_sc as plsc

_SIZE = 256
_NFACES = 256
_TEXW = 512
_P = _SIZE * _SIZE

_ROWS_PER_STEP = 32
_GRID = _SIZE // _ROWS_PER_STEP

_NW = 32
_PPW = _P // _NW
_CHUNK = 128
_NCHUNK = _PPW // _CHUNK


def _lane_shuffle(s, idx):
    return lax.gather(
        s, idx[:, None],
        dimension_numbers=lax.GatherDimensionNumbers(
            offset_dims=(), collapsed_slice_dims=(0,), start_index_map=(0,)),
        slice_sizes=(1,),
        mode=lax.GatherScatterMode.PROMISE_IN_BOUNDS)


def _area2d(ax, ay, bx, by, cx, cy):
    return (bx - ax) * (cy - ay) - (by - ay) * (cx - ax)


def _raster_body(td, nv, zinit, px_ref, py_ref,
                 idx0, w0, w1, w2, w3, alpha):
    px = px_ref[...]
    py = py_ref[...]
    shp = px.shape
    pinf = jnp.full(shp, jnp.inf, jnp.float32)
    zero = jnp.zeros(shp, jnp.float32)

    def body(f, carry):
        best_k, best_u, best_v = carry
        x0 = td[f, 0]
        y0 = td[f, 1]
        x1 = td[f, 2]
        y1 = td[f, 3]
        x2 = td[f, 4]
        y2 = td[f, 5]
        u0 = td[f, 6]
        u1 = td[f, 7]
        u2 = td[f, 8]
        v0 = td[f, 9]
        v1 = td[f, 10]
        v2 = td[f, 11]
        zi0 = td[f, 12]
        zi1 = td[f, 13]
        zi2 = td[f, 14]
        inv_a = td[f, 15]
        pab = (px - x1) * (y0 - y1) - (py - y1) * (x0 - x1)
        pcb = (px - x2) * (y1 - y2) - (py - y2) * (x1 - x2)
        pca = (px - x0) * (y2 - y0) - (py - y0) * (x2 - x0)
        cover = (jnp.maximum(pab, 0.0) * jnp.maximum(pcb, 0.0)
                 * jnp.maximum(pca, 0.0)) > 0
        bw1 = pcb * inv_a
        bw2 = pca * inv_a
        bw3 = 1.0 - bw1 - bw2
        zinv = bw1 * zi0 + bw2 * zi1 + bw3 * zi2
        key = jnp.where(cover, zinv, jnp.inf)
        ui = bw1 * u0 + bw2 * u1 + bw3 * u2
        vi = bw1 * v0 + bw2 * v1 + bw3 * v2
        upd = key <= best_k
        return (jnp.where(upd, key, best_k),
                jnp.where(upd, ui, best_u),
                jnp.where(upd, vi, best_v))

    best_k, best_u, best_v = lax.fori_loop(
        0, nv[0, 0], body, (pinf, zero, zero))

    hit = best_k < jnp.inf
    best_z = jnp.where(hit, 1.0 / jnp.where(hit, best_k, 1.0), -jnp.inf)
    write = best_z >= zinit[0, 0]
    denom = jnp.where(write, best_k, 1.0)
    ptsz = 1.0 / denom
    uf = best_u * ptsz
    vf = best_v * ptsz

    x = (uf + 1.0) * 512.0 / 2.0 - 0.5
    y = (vf + 1.0) * 512.0 / 2.0 - 0.5
    x0f = jnp.floor(x)
    y0f = jnp.floor(y)
    wx1 = x - x0f
    wx0 = 1.0 - wx1
    wy1 = y - y0f
    wy0 = 1.0 - wy1

    def tap(xi, yi, wt, w_ref):
        validt = ((xi >= 0) & (xi <= _TEXW - 1)
                  & (yi >= 0) & (yi <= _TEXW - 1))
        w_ref[...] = jnp.where(write, wt * validt.astype(jnp.float32), 0.0)

    tap(x0f, y0f, wx0 * wy0, w0)
    tap(x0f + 1.0, y0f, wx1 * wy0, w1)
    tap(x0f, y0f + 1.0, wx0 * wy1, w2)
    tap(x0f + 1.0, y0f + 1.0, wx1 * wy1, w3)

    bx = jnp.clip(x0f, -1.0, 511.0)
    by = jnp.clip(y0f, -1.0, 511.0)
    bxi = jnp.clip((bx + 1.0).astype(jnp.int32), 0, _TEXW)
    byi = jnp.clip((by + 1.0).astype(jnp.int32), 0, _TEXW)
    idx0[...] = byi * (_TEXW + 1) + bxi
    alpha[...] = write.astype(jnp.float32)


def _rasterize(td, nv, zinit, px, py):
    bspec = pl.BlockSpec((_ROWS_PER_STEP, _SIZE), lambda i: (i, 0))
    sspec = pl.BlockSpec(memory_space=pltpu.SMEM)
    i32 = jax.ShapeDtypeStruct((_SIZE, _SIZE), jnp.int32)
    f32 = jax.ShapeDtypeStruct((_SIZE, _SIZE), jnp.float32)
    return pl.pallas_call(
        _raster_body,
        grid=(_GRID,),
        in_specs=[sspec, sspec, sspec, bspec, bspec],
        out_specs=[bspec] * 6,
        out_shape=[i32, f32, f32, f32, f32, f32],
    )(td, nv, zinit, px, py)


_N1 = _TEXW + 1
_EXTW = _TEXW + 2
_EXTP = 520
_STG = 6 * _EXTP
_STGSZ = _STG + 16 + _N1 + 15


@functools.lru_cache(maxsize=1)
def _make_sc_texbuild():
    mesh = plsc.VectorSubcoreMesh(core_axis_name="c", subcore_axis_name="s",
                                  num_cores=2, num_subcores=16)

    @functools.partial(
        pl.kernel,
        out_type=jax.ShapeDtypeStruct((_N1 * _N1 * 16,), jnp.float32),
        mesh=mesh,
        scratch_types=[
            pltpu.VMEM((_STGSZ,), jnp.float32),
            pltpu.VMEM((_N1 * 16,), jnp.float32),
            pltpu.VMEM((_N1 * 16,), jnp.float32),
            pltpu.SemaphoreType.DMA,
        ],
        compiler_params=pltpu.CompilerParams(use_tc_tiling_on_sc=False,
                                             needs_layout_passes=False),
    )
    def texbuild(ext_hbm, out_hbm, stg, ob0, ob1, sem):
        wid = lax.axis_index("s") * 2 + lax.axis_index("c")
        y0 = wid * 16
        lane = lax.iota(jnp.int32, 16)
        grp = lax.shift_right_logical(lane, 2)
        ch = jax.lax.bitwise_and(lane, 3)
        dy = lax.shift_right_logical(grp, 1)
        dx = jax.lax.bitwise_and(grp, 1)
        offs = jnp.where(ch < 3, ch * (2 * _EXTP) + dy * _EXTP + dx, _STG)
        zero16 = jnp.zeros((16,), jnp.float32)
        for z in range(_STG, _STGSZ - 15, 16):
            stg[pl.ds(z, 16)] = zero16

        obufs = (ob0, ob1)
        handles = [None, None]
        for i in range(17):
            y = y0 + i
            ob = obufs[i % 2]
            if handles[i % 2] is not None:
                handles[i % 2].wait()
            for c in range(3):
                for d in range(2):
                    pltpu.sync_copy(
                        ext_hbm.at[c, y + d],
                        stg.at[pl.ds((c * 2 + d) * _EXTP, _EXTW)])

            def xbody(x, _):
                ob[pl.ds(x * 16, 16)] = plsc.load_gather(stg, [x + offs])
                return 0

            lax.fori_loop(0, _N1, xbody, 0)
            handles[i % 2] = pltpu.async_copy(
                ob, out_hbm.at[pl.ds(y * (_N1 * 16), _N1 * 16)], sem)
        for h in handles:
            h.wait()

    return texbuild


@functools.lru_cache(maxsize=1)
def _make_sc_sampler():
    mesh = plsc.VectorSubcoreMesh(core_axis_name="c", subcore_axis_name="s",
                                  num_cores=2, num_subcores=16)

    @functools.partial(
        pl.kernel,
        out_type=jax.ShapeDtypeStruct((3, _NW, _PPW), jnp.float32),
        mesh=mesh,
        scratch_types=[
            pltpu.VMEM((_NCHUNK, _CHUNK), jnp.int32),
            pltpu.VMEM((4 * _PPW,), jnp.float32),
            pltpu.VMEM((_PPW, 16), jnp.float32),
            pltpu.VMEM((3 * _PPW,), jnp.float32),
            pltpu.SemaphoreType.DMA,
        ],
        compiler_params=pltpu.CompilerParams(use_tc_tiling_on_sc=False,
                                             needs_layout_passes=False),
    )
    def sampler(tex_hbm, idx_hbm, w_hbm, out_hbm, idxs, ws, rows, outb, sem):
        wid = lax.axis_index("s") * 2 + lax.axis_index("c")
        pltpu.sync_copy(idx_hbm.at[wid], idxs)
        copies = []
        for c in range(_NCHUNK):
            copies.append(pltpu.async_copy(
                tex_hbm.at[idxs.at[c]],
                rows.at[pl.ds(c * _CHUNK, _CHUNK)], sem))
        pltpu.sync_copy(w_hbm.at[wid], ws)
        for cp in copies:
            cp.wait()

        lane = lax.iota(jnp.int32, 16)
        rot4 = lax.rem(lane + 4, 16)
        rot8 = lax.rem(lane + 8, 16)
        rot12 = lax.rem(lane + 12, 16)
        out_off = lane * _PPW
        out_mask = lane < 3
        woff = jax.lax.shift_right_logical(lane, 2) * _PPW

        def accum(i, _):
            for j in range(4):
                p = i * 4 + j
                s = rows[p, :] * plsc.load_gather(ws, [woff + p])
                r = s + _lane_shuffle(s, rot4)
                r = r + _lane_shuffle(s, rot8)
                r = r + _lane_shuffle(s, rot12)
                plsc.store_scatter(outb, [out_off + p], r, mask=out_mask)
            return 0

        lax.fori_loop(0, _PPW // 4, accum, 0)
        for ch in range(3):
            pltpu.sync_copy(outb.at[pl.ds(ch * _PPW, _PPW)],
                            out_hbm.at[ch, wid])

    return sampler


def kernel(vertices, faces, uv, uvfaces, uvmap):
    tris = vertices[faces]
    z_inv = 1.0 / tris[:, :, 2]
    uv2 = uv * 2.0 - 1.0
    uvs = uv2[uvfaces] * z_inv[..., None]
    t2 = tris[:, :, :2]
    area = _area2d(t2[:, 0, 0], t2[:, 0, 1], t2[:, 1, 0], t2[:, 1, 1],
                   t2[:, 2, 0], t2[:, 2, 1])
    normals = jnp.cross(tris[:, 1] - tris[:, 0], tris[:, 2] - tris[:, 0])
    normals = normals / jnp.linalg.norm(normals)
    bf = (normals @ jnp.array([0.0, 0.0, 1.0])) > 0
    valid = bf & (area >= 1e-9)
    a_s = jnp.where(jnp.abs(area) > 1e-12, area, 1.0)
    z_init = jnp.min(vertices[:, 2])

    td = jnp.stack([
        t2[:, 0, 0], t2[:, 0, 1], t2[:, 1, 0], t2[:, 1, 1],
        t2[:, 2, 0], t2[:, 2, 1],
        uvs[:, 0, 0], uvs[:, 1, 0], uvs[:, 2, 0],
        uvs[:, 0, 1], uvs[:, 1, 1], uvs[:, 2, 1],
        z_inv[:, 0], z_inv[:, 1], z_inv[:, 2],
        1.0 / a_s,
    ], axis=1)
    order = jnp.argsort(jnp.logical_not(valid), stable=True)
    td = td[order]
    nv = jnp.sum(valid.astype(jnp.int32)).reshape(1, 1)
    zinit = z_init.reshape(1, 1)

    lin = jnp.linspace(-1.0, 1.0, _SIZE)
    xx, yy = jnp.meshgrid(lin, lin, indexing='ij')
    pts = jnp.rot90(jnp.stack([xx, yy], axis=-1), 1)
    px = pts[..., 0]
    py = pts[..., 1]

    ib, w0, w1, w2, w3, alpha = _rasterize(td, nv, zinit, px, py)

    ext = jnp.pad(uvmap, ((0, 0), (1, 1), (1, 1)), mode='edge')
    tex = _make_sc_texbuild()(ext).reshape(_N1 * _N1, 16)

    idx_sc = ib.reshape(_NW, _NCHUNK, _CHUNK)
    w_sc = jnp.stack([w0.reshape(_NW, _PPW), w1.reshape(_NW, _PPW),
                      w2.reshape(_NW, _PPW), w3.reshape(_NW, _PPW)],
                     axis=1).reshape(_NW, 4 * _PPW)

    rgb = _make_sc_sampler()(tex, idx_sc, w_sc).reshape(3, _SIZE, _SIZE)
    return jnp.concatenate([rgb, alpha.reshape(1, _SIZE, _SIZE)], axis=0)
